# Initial kernel scaffold; baseline (speedup 1.0000x reference)
import jax
import jax.numpy as jnp
from jax.experimental import pallas as pl


def kernel(z, weight):
    raise NotImplementedError("write your pallas kernel here")


# speedup vs baseline: 1.0894x; 1.0894x over previous
import functools

import jax
import jax.numpy as jnp
from jax import lax
from jax.experimental import pallas as pl
from jax.experimental.pallas import tpu as pltpu
from jax.experimental.pallas import tpu_sc as plsc

_N_CODES = 8192
_DIM = 64
_BLK_M = 1024
_BLK_N = 2048


def _dist_argmin_body(z_ref, w_ref, idx_ref, loss_ref):
    i = pl.program_id(0)
    zb = z_ref[...]
    zsq = jnp.sum(zb * zb, axis=1, keepdims=True)
    zb2 = (2.0 * zb).astype(jnp.bfloat16)
    state = jnp.full((_BLK_M, 1), jnp.inf, jnp.float32)
    best_val = jnp.zeros((_BLK_M, 1), jnp.float32)
    best_idx = jnp.zeros((_BLK_M, 1), jnp.int32)
    for j in range(_N_CODES // _BLK_N):
        wj = w_ref[pl.ds(j * _BLK_N, _BLK_N), :]
        m2 = lax.dot_general(zb2, wj.astype(jnp.bfloat16),
                             (((1,), (1,)), ((), ())),
                             preferred_element_type=jnp.float32)
        wsq = jnp.sum(wj * wj, axis=1)
        d = (zsq - m2) + wsq[None, :]
        cmin = jnp.min(d, axis=1, keepdims=True)
        ids = lax.broadcasted_iota(jnp.int32, (_BLK_M, _BLK_N), 1) + (j * _BLK_N)
        carg = jnp.min(jnp.where(d == cmin, ids, jnp.int32(2**30)),
                       axis=1, keepdims=True)
        better = cmin < state
        state = jnp.where(better, cmin.astype(jnp.bfloat16).astype(jnp.float32),
                          state)
        best_val = jnp.where(better, cmin, best_val)
        best_idx = jnp.where(better, carg, best_idx)
    idx_ref[...] = best_idx

    @pl.when(i == 0)
    def _():
        loss_ref[...] = jnp.zeros((1, 1), jnp.float32)

    loss_ref[...] += jnp.sum(best_val, axis=(0, 1), keepdims=True)

    @pl.when(i == pl.num_programs(0) - 1)
    def _():
        loss_ref[...] = loss_ref[...] * (2.0 / (_N_CODES * _DIM))


def _dist_argmin(zf, weight, interpret=False):
    grid = (zf.shape[0] // _BLK_M,)
    return pl.pallas_call(
        _dist_argmin_body,
        grid=grid,
        in_specs=[
            pl.BlockSpec((_BLK_M, _DIM), lambda i: (i, 0)),
            pl.BlockSpec((_N_CODES, _DIM), lambda i: (0, 0)),
        ],
        out_specs=[
            pl.BlockSpec((_BLK_M, 1), lambda i: (i, 0)),
            pl.BlockSpec((1, 1), lambda i: (0, 0)),
        ],
        out_shape=[
            jax.ShapeDtypeStruct((zf.shape[0], 1), jnp.int32),
            jax.ShapeDtypeStruct((1, 1), jnp.float32),
        ],
        interpret=interpret,
    )(zf, weight)


def _sc_gather(weight, idx):
    info = plsc.get_sparse_core_info()
    nc, ns = info.num_cores, info.num_subcores
    nw = nc * ns
    b = idx.shape[0]
    b_per_w = b // nw
    ch = min(128, b_per_w)
    nch = b_per_w // ch
    mesh = plsc.VectorSubcoreMesh(core_axis_name="c", subcore_axis_name="s")

    @functools.partial(
        pl.kernel,
        out_type=jax.ShapeDtypeStruct((b, _DIM), jnp.float32),
        mesh=mesh,
        scratch_types=[
            pltpu.VMEM((ch,), jnp.int32),
            pltpu.VMEM((ch, _DIM), jnp.float32),
            pltpu.SemaphoreType.DMA,
        ],
        compiler_params=pltpu.CompilerParams(use_tc_tiling_on_sc=False),
    )
    def gather_kernel(table_hbm, idx_hbm, out_hbm, idx_v, rows_v, sem):
        wid = lax.axis_index("s") * nc + lax.axis_index("c")
        base = wid * b_per_w
        for j in range(nch):
            off = base + j * ch
            pltpu.sync_copy(idx_hbm.at[pl.ds(off, ch)], idx_v)
            pltpu.async_copy(table_hbm.at[idx_v], rows_v, sem).wait()
            pltpu.sync_copy(rows_v, out_hbm.at[pl.ds(off, ch)])

    return gather_kernel(weight, idx)


def kernel(z, weight):
    zf = z.reshape(-1, _DIM)
    idx2d, loss = _dist_argmin(zf, weight)
    idx = idx2d.reshape(zf.shape[0])
    code = _sc_gather(weight, idx)
    return (code.reshape(z.shape), loss[0, 0], idx.reshape(z.shape[:-1]))
